# baseline (speedup 1.0000x reference)
import functools

import jax
import jax.numpy as jnp
from jax import lax
from jax.experimental import pallas as pl
from jax.experimental.pallas import tpu as pltpu
from jax.experimental.pallas import tpu_sc as plsc

_D = 1024
_NC = 2
_NS = 16
_NW = _NC * _NS
_CH = 8
_NB = 13


def _make_gather(n_idx: int):
    per_w = n_idx // _NW
    nch = per_w // _CH
    mesh = plsc.VectorSubcoreMesh(core_axis_name="c", subcore_axis_name="s")

    @functools.partial(
        pl.kernel,
        mesh=mesh,
        out_type=jax.ShapeDtypeStruct((n_idx, _D), jnp.float32),
        scratch_types=(
            [pltpu.VMEM((nch, _CH), jnp.int32)]
            + [pltpu.VMEM((_CH, _D), jnp.float32)] * _NB
            + [pltpu.SemaphoreType.DMA] * (2 * _NB)
        ),
    )
    def gather_kernel(x_hbm, p2e_hbm, out_hbm, idx_v, *bufs):
        rows = bufs[:_NB]
        gsem = bufs[_NB:2 * _NB]
        wsem = bufs[2 * _NB:]
        wid = lax.axis_index("s") * _NC + lax.axis_index("c")
        base = wid * per_w
        pltpu.sync_copy(x_hbm.at[wid], idx_v)

        for j in range(_NB - 1):
            pltpu.async_copy(p2e_hbm.at[idx_v.at[j]], rows[j], gsem[j])
        for j in range(nch):
            b = j % _NB
            pltpu.make_async_copy(p2e_hbm.at[idx_v.at[j]], rows[b],
                                  gsem[b]).wait()
            pltpu.async_copy(rows[b],
                             out_hbm.at[pl.ds(base + j * _CH, _CH)],
                             wsem[b])
            nxt = j + _NB - 1
            if nxt < nch:
                nb2 = nxt % _NB
                if nxt >= _NB:
                    pltpu.make_async_copy(
                        rows[nb2],
                        out_hbm.at[pl.ds(base + (j - 1) * _CH, _CH)],
                        wsem[nb2]).wait()
                pltpu.async_copy(p2e_hbm.at[idx_v.at[nxt]], rows[nb2],
                                 gsem[nb2])
        for j in range(nch - _NB, nch):
            b = j % _NB
            pltpu.make_async_copy(
                rows[b], out_hbm.at[pl.ds(base + j * _CH, _CH)],
                wsem[b]).wait()

    return gather_kernel


def kernel(x, p2e):
    shp = x.shape
    n_idx = x.size
    x3 = x.reshape(_NW, (n_idx // _NW) // _CH, _CH)
    out = _make_gather(n_idx)(x3, p2e)
    return out.reshape(shp + (_D,))


# speedup vs baseline: 1.0207x; 1.0207x over previous
import functools

import jax
import jax.numpy as jnp
from jax import lax
from jax.experimental import pallas as pl
from jax.experimental.pallas import tpu as pltpu
from jax.experimental.pallas import tpu_sc as plsc

_D = 1024
_NC = 2
_NS = 16
_NW = _NC * _NS
_CH = 16
_NB = 7


def _make_gather(n_idx: int):
    per_w = n_idx // _NW
    nch = per_w // _CH
    mesh = plsc.VectorSubcoreMesh(core_axis_name="c", subcore_axis_name="s")

    @functools.partial(
        pl.kernel,
        mesh=mesh,
        out_type=jax.ShapeDtypeStruct((n_idx, _D), jnp.float32),
        scratch_types=(
            [pltpu.VMEM((nch, _CH), jnp.int32)]
            + [pltpu.VMEM((_CH, _D), jnp.float32)] * _NB
            + [pltpu.SemaphoreType.DMA] * (2 * _NB)
        ),
    )
    def gather_kernel(x_hbm, p2e_hbm, out_hbm, idx_v, *bufs):
        rows = bufs[:_NB]
        gsem = bufs[_NB:2 * _NB]
        wsem = bufs[2 * _NB:]
        wid = lax.axis_index("s") * _NC + lax.axis_index("c")
        base = wid * per_w
        pltpu.sync_copy(x_hbm.at[wid], idx_v)

        for j in range(_NB - 1):
            pltpu.async_copy(p2e_hbm.at[idx_v.at[j]], rows[j], gsem[j])
        for j in range(nch):
            b = j % _NB
            pltpu.make_async_copy(p2e_hbm.at[idx_v.at[j]], rows[b],
                                  gsem[b]).wait()
            pltpu.async_copy(rows[b],
                             out_hbm.at[pl.ds(base + j * _CH, _CH)],
                             wsem[b])
            nxt = j + _NB - 1
            if nxt < nch:
                nb2 = nxt % _NB
                if nxt >= _NB:
                    pltpu.make_async_copy(
                        rows[nb2],
                        out_hbm.at[pl.ds(base + (j - 1) * _CH, _CH)],
                        wsem[nb2]).wait()
                pltpu.async_copy(p2e_hbm.at[idx_v.at[nxt]], rows[nb2],
                                 gsem[nb2])
        for j in range(nch - _NB, nch):
            b = j % _NB
            pltpu.make_async_copy(
                rows[b], out_hbm.at[pl.ds(base + j * _CH, _CH)],
                wsem[b]).wait()

    return gather_kernel


def kernel(x, p2e):
    shp = x.shape
    n_idx = x.size
    x3 = x.reshape(_NW, (n_idx // _NW) // _CH, _CH)
    out = _make_gather(n_idx)(x3, p2e)
    return out.reshape(shp + (_D,))
